# initial kernel scaffold (unmeasured)
import jax
import jax.numpy as jnp
from jax import lax
from jax.experimental import pallas as pl
from jax.experimental.pallas import tpu as pltpu


def kernel(
    x,
):
    def body(*refs):
        pass

    out_shape = jax.ShapeDtypeStruct(..., jnp.float32)
    return pl.pallas_call(body, out_shape=out_shape)(...)


# baseline (device time: 9545 ns/iter reference)
import jax
import jax.numpy as jnp
from jax import lax
from jax.experimental import pallas as pl
from jax.experimental.pallas import tpu as pltpu


def kernel(x):
    _, m, n = x.shape

    def body(x_ref, out_ref, stage, recv_x, recv_y, recv_d, send_sems, recv_sems):
        my_x = lax.axis_index("x")
        my_y = lax.axis_index("y")
        ox = 1 - my_x
        oy = 1 - my_y

        barrier = pltpu.get_barrier_semaphore()
        for tgt in [(ox, my_y), (my_x, oy), (ox, oy)]:
            pl.semaphore_signal(
                barrier, inc=1, device_id=tgt,
                device_id_type=pl.DeviceIdType.MESH,
            )
        pl.semaphore_wait(barrier, 3)

        stage[...] = x_ref[0].astype(jnp.bfloat16)

        rx = pltpu.make_async_remote_copy(
            src_ref=stage, dst_ref=recv_x,
            send_sem=send_sems.at[0], recv_sem=recv_sems.at[0],
            device_id=(ox, my_y), device_id_type=pl.DeviceIdType.MESH,
        )
        ry = pltpu.make_async_remote_copy(
            src_ref=stage, dst_ref=recv_y,
            send_sem=send_sems.at[1], recv_sem=recv_sems.at[1],
            device_id=(my_x, oy), device_id_type=pl.DeviceIdType.MESH,
        )
        rd = pltpu.make_async_remote_copy(
            src_ref=stage, dst_ref=recv_d,
            send_sem=send_sems.at[2], recv_sem=recv_sems.at[2],
            device_id=(ox, oy), device_id_type=pl.DeviceIdType.MESH,
        )
        rx.start()
        ry.start()
        rd.start()

        rx.wait_recv()
        own = x_ref[0] + recv_x[...].astype(jnp.float32)
        ry.wait_recv()
        rd.wait_recv()
        other = recv_y[...].astype(jnp.float32) + recv_d[...].astype(jnp.float32)

        @pl.when(my_y == 0)
        def _():
            out_ref[:, :n] = own
            out_ref[:, n:] = other

        @pl.when(my_y == 1)
        def _():
            out_ref[:, :n] = other
            out_ref[:, n:] = own

        rx.wait_send()
        ry.wait_send()
        rd.wait_send()

    out_shape = jax.ShapeDtypeStruct((m, 2 * n), jnp.float32)
    return pl.pallas_call(
        body,
        out_shape=out_shape,
        in_specs=[pl.BlockSpec(memory_space=pltpu.VMEM)],
        out_specs=pl.BlockSpec(memory_space=pltpu.VMEM),
        scratch_shapes=[
            pltpu.VMEM((m, n), jnp.bfloat16),
            pltpu.VMEM((m, n), jnp.bfloat16),
            pltpu.VMEM((m, n), jnp.bfloat16),
            pltpu.VMEM((m, n), jnp.bfloat16),
            pltpu.SemaphoreType.DMA((3,)),
            pltpu.SemaphoreType.DMA((3,)),
        ],
        compiler_params=pltpu.CompilerParams(collective_id=0),
    )(x)
